# Initial kernel scaffold; baseline (speedup 1.0000x reference)
import jax
import jax.numpy as jnp
from jax.experimental import pallas as pl


def kernel(node_ids, top_mask, edge_ids, node_table, pos_table, edge_table, ln_gamma, ln_beta):
    raise NotImplementedError("write your pallas kernel here")


# speedup vs baseline: 2.3863x; 2.3863x over previous
import functools

import jax
import jax.numpy as jnp
from jax import lax
from jax.experimental import pallas as pl
from jax.experimental.pallas import tpu as pltpu
from jax.experimental.pallas import tpu_sc as plsc

N_NODES = 10000
N_EDGES = 160000
HID = 256
POS_VOCAB = 10000

NC = 2
NS = 16
NW = NC * NS
L = 16

C = 320
NP = NW * C
SUB = 80
NSUB = C // SUB
ZROW = POS_VOCAB

BE = 3200
NB = N_EDGES // BE


def _node_body(ids_hbm, mask_hbm, ntab_hbm, ptab_hbm, out_hbm,
               ids_v, mask_v, pidx_v, nbuf, pbuf, sem_n, sem_p):
    cid = lax.axis_index("c")
    sid = lax.axis_index("s")
    wid = sid * NC + cid
    base = wid * C

    pltpu.sync_copy(mask_hbm, mask_v)
    pltpu.sync_copy(ids_hbm.at[pl.ds(wid * NSUB, NSUB)], ids_v)

    def pre_body(k, acc):
        m = mask_v[pl.ds(k * L, L)]
        return acc + jnp.where(m == 1, 1, 0)

    accv = lax.fori_loop(0, base // L, pre_body, jnp.zeros((L,), jnp.int32))
    prefix = jnp.sum(accv)

    carry = jnp.zeros((), jnp.int32)
    for k in range(C // L):
        m = mask_v[pl.ds(base + k * L, L)]
        mi = jnp.where(m == 1, 1, 0)
        cs = plsc.cumsum(mi)
        rank = prefix + carry + (cs - mi)
        pidx = jnp.where(m == 1, rank, ZROW)
        r, c = divmod(k, SUB // L)
        pidx_v[r, pl.ds(c * L, L)] = pidx
        carry = carry + jnp.sum(mi)

    for s in range(NSUB):
        cp_n = pltpu.async_copy(ntab_hbm.at[ids_v.at[s]], nbuf, sem_n)
        cp_p = pltpu.async_copy(ptab_hbm.at[pidx_v.at[s]], pbuf, sem_p)
        cp_n.wait()
        cp_p.wait()

        def add_body(r, _):
            for c in range(HID // L):
                nbuf[r, pl.ds(c * L, L)] = (
                    nbuf[r, pl.ds(c * L, L)] + pbuf[r, pl.ds(c * L, L)])
            return 0

        lax.fori_loop(0, SUB, add_body, 0)
        pltpu.sync_copy(nbuf, out_hbm.at[pl.ds(base + s * SUB, SUB)])


def _edge_body(ids_ref, tbl_ref, g_ref, b_ref, out_ref):
    tbl = tbl_ref[...]
    mean = jnp.mean(tbl, axis=1, keepdims=True)
    var = jnp.mean((tbl - mean) ** 2, axis=1, keepdims=True)
    norm = (tbl - mean) * lax.rsqrt(var + 1e-5) * g_ref[...] + b_ref[...]
    ids = ids_ref[0, 0, :]
    onehot = (ids[:, None] == lax.broadcasted_iota(jnp.int32, (BE, 64), 1))
    out_ref[...] = jnp.dot(onehot.astype(jnp.float32), norm,
                           preferred_element_type=jnp.float32,
                           precision=lax.Precision.HIGHEST)


@jax.jit
def kernel(node_ids, top_mask, edge_ids, node_table, pos_table, edge_table,
           ln_gamma, ln_beta):
    node_ids = node_ids.astype(jnp.int32)
    ids2d = jnp.pad(node_ids, (0, NP - N_NODES)).reshape(NP // SUB, SUB)
    mask_pad = jnp.pad(top_mask.astype(jnp.int32), (0, NP - N_NODES))
    pos_aug = jnp.concatenate(
        [pos_table, jnp.zeros((8, HID), jnp.float32)], axis=0)

    node_out = pl.kernel(
        _node_body,
        out_type=jax.ShapeDtypeStruct((NP, HID), jnp.float32),
        mesh=plsc.VectorSubcoreMesh(core_axis_name="c", subcore_axis_name="s",
                                    num_cores=NC, num_subcores=NS),
        scratch_types=[
            pltpu.VMEM((NSUB, SUB), jnp.int32),
            pltpu.VMEM((NP,), jnp.int32),
            pltpu.VMEM((NSUB, SUB), jnp.int32),
            pltpu.VMEM((SUB, HID), jnp.float32),
            pltpu.VMEM((SUB, HID), jnp.float32),
            pltpu.SemaphoreType.DMA,
            pltpu.SemaphoreType.DMA,
        ],
        compiler_params=pltpu.CompilerParams(needs_layout_passes=False),
    )(ids2d, mask_pad, node_table, pos_aug)
    node_feat = node_out[:N_NODES]

    edge_feat = pl.pallas_call(
        _edge_body,
        grid=(NB,),
        in_specs=[
            pl.BlockSpec((1, 1, BE), lambda i: (i, 0, 0)),
            pl.BlockSpec((64, HID), lambda i: (0, 0)),
            pl.BlockSpec((1, HID), lambda i: (0, 0)),
            pl.BlockSpec((1, HID), lambda i: (0, 0)),
        ],
        out_specs=pl.BlockSpec((BE, HID), lambda i: (i, 0)),
        out_shape=jax.ShapeDtypeStruct((N_EDGES, HID), jnp.float32),
    )(edge_ids.reshape(NB, 1, BE), edge_table,
      ln_gamma.reshape(1, HID), ln_beta.reshape(1, HID))

    return node_feat, edge_feat


# speedup vs baseline: 2.3881x; 1.0008x over previous
import functools

import jax
import jax.numpy as jnp
from jax import lax
from jax.experimental import pallas as pl
from jax.experimental.pallas import tpu as pltpu
from jax.experimental.pallas import tpu_sc as plsc

N_NODES = 10000
N_EDGES = 160000
HID = 256
POS_VOCAB = 10000

NC = 2
NS = 16
NW = NC * NS
L = 16

C = 320
NP = NW * C
SUB = 80
NSUB = C // SUB
ZROW = POS_VOCAB

BE = 3200
NB = N_EDGES // BE


def _node_body(ids_hbm, mask_hbm, ntab_hbm, ptab_hbm, out_hbm,
               ids_v, mask_v, pidx_v, nbuf, pbuf, sem_n, sem_p):
    cid = lax.axis_index("c")
    sid = lax.axis_index("s")
    wid = sid * NC + cid
    base = wid * C

    pltpu.sync_copy(mask_hbm, mask_v)
    pltpu.sync_copy(ids_hbm.at[pl.ds(wid * NSUB, NSUB)], ids_v)

    def pre_body(k, acc):
        m = mask_v[pl.ds(k * L, L)]
        return acc + jnp.where(m == 1, 1, 0)

    accv = lax.fori_loop(0, base // L, pre_body, jnp.zeros((L,), jnp.int32))
    prefix = jnp.sum(accv)

    carry = jnp.zeros((), jnp.int32)
    for k in range(C // L):
        m = mask_v[pl.ds(base + k * L, L)]
        mi = jnp.where(m == 1, 1, 0)
        cs = plsc.cumsum(mi)
        rank = prefix + carry + (cs - mi)
        pidx = jnp.where(m == 1, rank, ZROW)
        r, c = divmod(k, SUB // L)
        pidx_v[r, pl.ds(c * L, L)] = pidx
        carry = carry + jnp.sum(mi)

    for s in range(NSUB):
        cp_n = pltpu.async_copy(ntab_hbm.at[ids_v.at[s]], nbuf, sem_n)
        cp_p = pltpu.async_copy(ptab_hbm.at[pidx_v.at[s]], pbuf, sem_p)
        cp_n.wait()
        cp_p.wait()

        def add_body(r, _):
            for c in range(HID // L):
                nbuf[r, pl.ds(c * L, L)] = (
                    nbuf[r, pl.ds(c * L, L)] + pbuf[r, pl.ds(c * L, L)])
            return 0

        lax.fori_loop(0, SUB, add_body, 0)
        pltpu.sync_copy(nbuf, out_hbm.at[pl.ds(base + s * SUB, SUB)])


def _edge_body(ids_ref, tbl_ref, g_ref, b_ref, out_ref):
    tbl = tbl_ref[...]
    mean = jnp.mean(tbl, axis=1, keepdims=True)
    var = jnp.mean((tbl - mean) ** 2, axis=1, keepdims=True)
    norm = (tbl - mean) * lax.rsqrt(var + 1e-5) * g_ref[...] + b_ref[...]
    norm_hi = norm.astype(jnp.bfloat16)
    norm_lo = (norm - norm_hi.astype(jnp.float32)).astype(jnp.bfloat16)
    ids = ids_ref[0, 0, :]
    onehot = (ids[:, None] == lax.broadcasted_iota(jnp.int32, (BE, 64), 1)
              ).astype(jnp.bfloat16)
    out_ref[...] = (
        jnp.dot(onehot, norm_hi, preferred_element_type=jnp.float32)
        + jnp.dot(onehot, norm_lo, preferred_element_type=jnp.float32))


@jax.jit
def kernel(node_ids, top_mask, edge_ids, node_table, pos_table, edge_table,
           ln_gamma, ln_beta):
    node_ids = node_ids.astype(jnp.int32)
    ids2d = jnp.pad(node_ids, (0, NP - N_NODES)).reshape(NP // SUB, SUB)
    mask_pad = jnp.pad(top_mask.astype(jnp.int32), (0, NP - N_NODES))
    pos_aug = jnp.concatenate(
        [pos_table, jnp.zeros((8, HID), jnp.float32)], axis=0)

    node_out = pl.kernel(
        _node_body,
        out_type=jax.ShapeDtypeStruct((NP, HID), jnp.float32),
        mesh=plsc.VectorSubcoreMesh(core_axis_name="c", subcore_axis_name="s",
                                    num_cores=NC, num_subcores=NS),
        scratch_types=[
            pltpu.VMEM((NSUB, SUB), jnp.int32),
            pltpu.VMEM((NP,), jnp.int32),
            pltpu.VMEM((NSUB, SUB), jnp.int32),
            pltpu.VMEM((SUB, HID), jnp.float32),
            pltpu.VMEM((SUB, HID), jnp.float32),
            pltpu.SemaphoreType.DMA,
            pltpu.SemaphoreType.DMA,
        ],
        compiler_params=pltpu.CompilerParams(needs_layout_passes=False),
    )(ids2d, mask_pad, node_table, pos_aug)
    node_feat = node_out[:N_NODES]

    edge_feat = pl.pallas_call(
        _edge_body,
        grid=(NB,),
        in_specs=[
            pl.BlockSpec((1, 1, BE), lambda i: (i, 0, 0)),
            pl.BlockSpec((64, HID), lambda i: (0, 0)),
            pl.BlockSpec((1, HID), lambda i: (0, 0)),
            pl.BlockSpec((1, HID), lambda i: (0, 0)),
        ],
        out_specs=pl.BlockSpec((BE, HID), lambda i: (i, 0)),
        out_shape=jax.ShapeDtypeStruct((N_EDGES, HID), jnp.float32),
    )(edge_ids.reshape(NB, 1, BE), edge_table,
      ln_gamma.reshape(1, HID), ln_beta.reshape(1, HID))

    return node_feat, edge_feat


# speedup vs baseline: 2.4244x; 1.0152x over previous
import functools

import jax
import jax.numpy as jnp
from jax import lax
from jax.experimental import pallas as pl
from jax.experimental.pallas import tpu as pltpu
from jax.experimental.pallas import tpu_sc as plsc

N_NODES = 10000
N_EDGES = 160000
HID = 256
POS_VOCAB = 10000

NC = 2
NS = 16
NW = NC * NS
L = 16

C = 320
NP = NW * C
SUB = 80
NSUB = C // SUB
ZROW = POS_VOCAB

BE = 3200
NB = N_EDGES // BE


def _node_body(ids_hbm, mask_hbm, ntab_hbm, ptab_hbm, out_hbm,
               ids_v, mask_v, pidx_v, nbufs, pbufs, sems_n, sems_p, sems_w):
    cid = lax.axis_index("c")
    sid = lax.axis_index("s")
    wid = sid * NC + cid
    base = wid * C

    with jax.named_scope("nk_stage"):
        cp_ids = pltpu.async_copy(ids_hbm.at[pl.ds(wid * NSUB, NSUB)], ids_v,
                                  sems_w[0])
        cp_mask = pltpu.async_copy(mask_hbm, mask_v, sems_w[1])
        cp_ids.wait()

    gn = [pltpu.async_copy(ntab_hbm.at[ids_v.at[s]], nbufs[s % 2], sems_n[s % 2])
          for s in range(2)]
    cp_mask.wait()

    with jax.named_scope("nk_prefix"):
        def pre_body(k, acc):
            m = mask_v[pl.ds(k * L, L)]
            return acc + jnp.where(m == 1, 1, 0)

        accv = lax.fori_loop(0, base // L, pre_body, jnp.zeros((L,), jnp.int32))
        prefix = jnp.sum(accv)

    with jax.named_scope("nk_ranks"):
        carry = jnp.zeros((), jnp.int32)
        for k in range(C // L):
            m = mask_v[pl.ds(base + k * L, L)]
            mi = jnp.where(m == 1, 1, 0)
            cs = plsc.cumsum(mi)
            rank = prefix + carry + (cs - mi)
            pidx = jnp.where(m == 1, rank, ZROW)
            r, c = divmod(k, SUB // L)
            pidx_v[r, pl.ds(c * L, L)] = pidx
            carry = carry + jnp.sum(mi)

    gp = [pltpu.async_copy(ptab_hbm.at[pidx_v.at[s]], pbufs[s % 2], sems_p[s % 2])
          for s in range(2)]

    wr = [None, None]
    for s in range(NSUB):
        b = s % 2
        with jax.named_scope("nk_wait"):
            gn[s].wait()
            gp[s].wait()
        with jax.named_scope("nk_add"):
            def add_body(r, _, nb=nbufs[b], pb=pbufs[b]):
                for c in range(HID // L):
                    nb[r, pl.ds(c * L, L)] = (
                        nb[r, pl.ds(c * L, L)] + pb[r, pl.ds(c * L, L)])
                return 0

            lax.fori_loop(0, SUB, add_body, 0)
        wr[b] = pltpu.async_copy(
            nbufs[b], out_hbm.at[pl.ds(base + s * SUB, SUB)], sems_w[b])
        if s + 2 < NSUB:
            with jax.named_scope("nk_refill"):
                wr[b].wait()
                gn.append(pltpu.async_copy(ntab_hbm.at[ids_v.at[s + 2]],
                                           nbufs[b], sems_n[b]))
                gp.append(pltpu.async_copy(ptab_hbm.at[pidx_v.at[s + 2]],
                                           pbufs[b], sems_p[b]))
    with jax.named_scope("nk_drain"):
        wr[0].wait()
        wr[1].wait()


def _edge_body(ids_ref, tbl_ref, g_ref, b_ref, out_ref):
    tbl = tbl_ref[...]
    mean = jnp.mean(tbl, axis=1, keepdims=True)
    var = jnp.mean((tbl - mean) ** 2, axis=1, keepdims=True)
    norm = (tbl - mean) * lax.rsqrt(var + 1e-5) * g_ref[...] + b_ref[...]
    norm_hi = norm.astype(jnp.bfloat16)
    norm_lo = (norm - norm_hi.astype(jnp.float32)).astype(jnp.bfloat16)
    ids = ids_ref[0, 0, :]
    onehot = (ids[:, None] == lax.broadcasted_iota(jnp.int32, (BE, 64), 1)
              ).astype(jnp.bfloat16)
    out_ref[...] = (
        jnp.dot(onehot, norm_hi, preferred_element_type=jnp.float32)
        + jnp.dot(onehot, norm_lo, preferred_element_type=jnp.float32))


@jax.jit
def kernel(node_ids, top_mask, edge_ids, node_table, pos_table, edge_table,
           ln_gamma, ln_beta):
    node_ids = node_ids.astype(jnp.int32)
    ids2d = jnp.pad(node_ids, (0, NP - N_NODES)).reshape(NP // SUB, SUB)
    mask_pad = jnp.pad(top_mask.astype(jnp.int32), (0, NP - N_NODES))
    pos_aug = jnp.concatenate(
        [pos_table, jnp.zeros((8, HID), jnp.float32)], axis=0)

    node_out = pl.kernel(
        _node_body,
        out_type=jax.ShapeDtypeStruct((NP, HID), jnp.float32),
        mesh=plsc.VectorSubcoreMesh(core_axis_name="c", subcore_axis_name="s",
                                    num_cores=NC, num_subcores=NS),
        scratch_types=[
            pltpu.VMEM((NSUB, SUB), jnp.int32),
            pltpu.VMEM((NP,), jnp.int32),
            pltpu.VMEM((NSUB, SUB), jnp.int32),
            [pltpu.VMEM((SUB, HID), jnp.float32)] * 2,
            [pltpu.VMEM((SUB, HID), jnp.float32)] * 2,
            [pltpu.SemaphoreType.DMA] * 2,
            [pltpu.SemaphoreType.DMA] * 2,
            [pltpu.SemaphoreType.DMA] * 2,
        ],
        compiler_params=pltpu.CompilerParams(needs_layout_passes=False),
    )(ids2d, mask_pad, node_table, pos_aug)
    node_feat = node_out[:N_NODES]

    edge_feat = pl.pallas_call(
        _edge_body,
        grid=(NB,),
        in_specs=[
            pl.BlockSpec((1, 1, BE), lambda i: (i, 0, 0)),
            pl.BlockSpec((64, HID), lambda i: (0, 0)),
            pl.BlockSpec((1, HID), lambda i: (0, 0)),
            pl.BlockSpec((1, HID), lambda i: (0, 0)),
        ],
        out_specs=pl.BlockSpec((BE, HID), lambda i: (i, 0)),
        out_shape=jax.ShapeDtypeStruct((N_EDGES, HID), jnp.float32),
    )(edge_ids.reshape(NB, 1, BE), edge_table,
      ln_gamma.reshape(1, HID), ln_beta.reshape(1, HID))

    return node_feat, edge_feat


# speedup vs baseline: 6.6789x; 2.7548x over previous
import jax
import jax.numpy as jnp
from jax import lax
from jax.experimental import pallas as pl
from jax.experimental.pallas import tpu as pltpu
from jax.experimental.pallas import tpu_sc as plsc

N_NODES = 10000
N_EDGES = 160000
HID = 256
POS_VOCAB = 10000

NC = 2
NS = 16
NW = NC * NS
L = 16

C = 320
NP = NW * C
RPS = 40
NSTR = C // RPS
SUB = 80
NSUB = C // SUB
PWIN = SUB + 8

BE = 3200
NB = N_EDGES // BE


def _node_body(ids_hbm, mask_hbm, ntab_hbm, ptab_hbm, out_hbm,
               ids_v, mask_v, rank_v, nbufs, pbuf,
               sems_n, sem_p, sems_w, sem_s):
    cid = lax.axis_index("c")
    sid = lax.axis_index("s")
    wid = sid * NC + cid
    base = wid * C

    with jax.named_scope("nk_stage"):
        pltpu.async_copy(ids_hbm.at[pl.ds(wid * NSTR, NSTR)], ids_v,
                         sem_s).wait()
    gn = [pltpu.async_copy(ntab_hbm.at[ids_v.at[j]],
                           nbufs[j // 2].at[pl.ds((j % 2) * RPS, RPS)],
                           sems_n[j // 2])
          for j in range(NSTR)]
    cp_mask = pltpu.async_copy(mask_hbm, mask_v.at[pl.ds(0, NP)], sem_s)
    cp_mask.wait()

    with jax.named_scope("nk_prefix"):
        def pre_body(k, acc):
            m = mask_v[pl.ds(k * L, L)]
            return acc + jnp.where(m == 1, 1, 0)

        accv = lax.fori_loop(0, base // L, pre_body, jnp.zeros((L,), jnp.int32))
        prefix = jnp.sum(accv)

    with jax.named_scope("nk_ranks"):
        starts = []
        carry = jnp.zeros((), jnp.int32)
        for k in range(C // L):
            if k % (SUB // L) == 0:
                st = jnp.minimum(prefix + carry, POS_VOCAB - PWIN)
                starts.append(pl.multiple_of((st // 8) * 8, 8))
            m = mask_v[pl.ds(base + k * L, L)]
            mi = jnp.where(m == 1, 1, 0)
            cs = plsc.cumsum(mi)
            rank_v[pl.ds(k * L, L)] = prefix + carry + (cs - mi)
            carry = carry + jnp.sum(mi)

    gp = pltpu.async_copy(ptab_hbm.at[pl.ds(starts[0], PWIN)], pbuf, sem_p)

    wr = []
    for s in range(NSUB):
        with jax.named_scope("nk_wait"):
            gn[2 * s].wait()
            gn[2 * s + 1].wait()
            gp.wait()
        with jax.named_scope("nk_add"):
            def add_body(j, _, nb=nbufs[s], st=starts[s], s=s):
                m = mask_v[pl.ds(base + s * SUB + j, L)][0]

                @pl.when(m == 1)
                def _masked_add():
                    rel = rank_v[pl.ds(s * SUB + j, L)][0] - st
                    for cb in range(HID // L):
                        nb[j, pl.ds(cb * L, L)] = (
                            nb[j, pl.ds(cb * L, L)]
                            + pbuf[rel, pl.ds(cb * L, L)])

                return 0

            lax.fori_loop(0, SUB, add_body, 0)
        wr.append(pltpu.async_copy(
            nbufs[s], out_hbm.at[pl.ds(base + s * SUB, SUB)], sems_w[s]))
        if s + 1 < NSUB:
            gp = pltpu.async_copy(ptab_hbm.at[pl.ds(starts[s + 1], PWIN)],
                                  pbuf, sem_p)
    with jax.named_scope("nk_drain"):
        for w in wr:
            w.wait()


def _edge_body(ids_ref, tbl_ref, g_ref, b_ref, out_ref):
    tbl = tbl_ref[...]
    mean = jnp.mean(tbl, axis=1, keepdims=True)
    var = jnp.mean((tbl - mean) ** 2, axis=1, keepdims=True)
    norm = (tbl - mean) * lax.rsqrt(var + 1e-5) * g_ref[...] + b_ref[...]
    norm_hi = norm.astype(jnp.bfloat16)
    norm_lo = (norm - norm_hi.astype(jnp.float32)).astype(jnp.bfloat16)
    ids = ids_ref[0, 0, :]
    onehot = (ids[:, None] == lax.broadcasted_iota(jnp.int32, (BE, 64), 1)
              ).astype(jnp.bfloat16)
    out_ref[...] = (
        jnp.dot(onehot, norm_hi, preferred_element_type=jnp.float32)
        + jnp.dot(onehot, norm_lo, preferred_element_type=jnp.float32))


@jax.jit
def kernel(node_ids, top_mask, edge_ids, node_table, pos_table, edge_table,
           ln_gamma, ln_beta):
    node_ids = node_ids.astype(jnp.int32)
    ids2d = jnp.pad(node_ids, (0, NP - N_NODES)).reshape(NP // RPS, RPS)
    mask_pad = jnp.pad(top_mask.astype(jnp.int32), (0, NP - N_NODES))

    node_out = pl.kernel(
        _node_body,
        out_type=jax.ShapeDtypeStruct((NP, HID), jnp.float32),
        mesh=plsc.VectorSubcoreMesh(core_axis_name="c", subcore_axis_name="s",
                                    num_cores=NC, num_subcores=NS),
        scratch_types=[
            pltpu.VMEM((NSTR, RPS), jnp.int32),
            pltpu.VMEM((NP + L,), jnp.int32),
            pltpu.VMEM((C + L,), jnp.int32),
            [pltpu.VMEM((SUB, HID), jnp.float32)] * NSUB,
            pltpu.VMEM((PWIN, HID), jnp.float32),
            [pltpu.SemaphoreType.DMA] * NSUB,
            pltpu.SemaphoreType.DMA,
            [pltpu.SemaphoreType.DMA] * NSUB,
            pltpu.SemaphoreType.DMA,
        ],
        compiler_params=pltpu.CompilerParams(needs_layout_passes=False),
    )(ids2d, mask_pad, node_table, pos_table)
    node_feat = node_out[:N_NODES]

    edge_feat = pl.pallas_call(
        _edge_body,
        grid=(NB,),
        in_specs=[
            pl.BlockSpec((1, 1, BE), lambda i: (i, 0, 0)),
            pl.BlockSpec((64, HID), lambda i: (0, 0)),
            pl.BlockSpec((1, HID), lambda i: (0, 0)),
            pl.BlockSpec((1, HID), lambda i: (0, 0)),
        ],
        out_specs=pl.BlockSpec((BE, HID), lambda i: (i, 0)),
        out_shape=jax.ShapeDtypeStruct((N_EDGES, HID), jnp.float32),
    )(edge_ids.reshape(NB, 1, BE), edge_table,
      ln_gamma.reshape(1, HID), ln_beta.reshape(1, HID))

    return node_feat, edge_feat


# speedup vs baseline: 6.7829x; 1.0156x over previous
import jax
import jax.numpy as jnp
from jax import lax
from jax.experimental import pallas as pl
from jax.experimental.pallas import tpu as pltpu
from jax.experimental.pallas import tpu_sc as plsc

N_NODES = 10000
N_EDGES = 160000
HID = 256
POS_VOCAB = 10000

NC = 2
NS = 16
NW = NC * NS
L = 16

C = 320
NP = NW * C
RPS = 40
NSTR = C // RPS
SUB = 80
NSUB = C // SUB
PWIN = SUB + 8

BE = 16000
NB = N_EDGES // BE


def _node_body(ids_hbm, mask_hbm, ntab_hbm, ptab_hbm, out_hbm,
               ids_v, mask_v, rank_v, nbufs, pbuf,
               sems_n, sem_p, sems_w, sem_s):
    cid = lax.axis_index("c")
    sid = lax.axis_index("s")
    wid = sid * NC + cid
    base = wid * C

    with jax.named_scope("nk_stage"):
        pltpu.async_copy(ids_hbm.at[pl.ds(wid * NSTR, NSTR)], ids_v,
                         sem_s).wait()
    gn = [pltpu.async_copy(ntab_hbm.at[ids_v.at[j]],
                           nbufs[j // 2].at[pl.ds((j % 2) * RPS, RPS)],
                           sems_n[j // 2])
          for j in range(NSTR)]
    cp_mask = pltpu.async_copy(mask_hbm, mask_v.at[pl.ds(0, NP)], sem_s)
    cp_mask.wait()

    with jax.named_scope("nk_prefix"):
        def pre_body(k, acc):
            m = mask_v[pl.ds(k * L, L)]
            return acc + jnp.where(m == 1, 1, 0)

        accv = lax.fori_loop(0, base // L, pre_body, jnp.zeros((L,), jnp.int32))
        prefix = jnp.sum(accv)

    with jax.named_scope("nk_ranks"):
        starts = []
        carry = jnp.zeros((), jnp.int32)
        for k in range(C // L):
            if k % (SUB // L) == 0:
                st = jnp.minimum(prefix + carry, POS_VOCAB - PWIN)
                starts.append(pl.multiple_of((st // 8) * 8, 8))
            m = mask_v[pl.ds(base + k * L, L)]
            mi = jnp.where(m == 1, 1, 0)
            cs = plsc.cumsum(mi)
            rank_v[pl.ds(k * L, L)] = prefix + carry + (cs - mi)
            carry = carry + jnp.sum(mi)

    gp = pltpu.async_copy(ptab_hbm.at[pl.ds(starts[0], PWIN)], pbuf, sem_p)

    for s in range(NSUB):
        with jax.named_scope("nk_wait"):
            gn[2 * s].wait()
            gn[2 * s + 1].wait()
            gp.wait()
        with jax.named_scope("nk_add"):
            def add_body(j, _, nb=nbufs[s], st=starts[s], s=s):
                m = mask_v[pl.ds(base + s * SUB + j, L)][0]

                @pl.when(m == 1)
                def _masked_add():
                    rel = rank_v[pl.ds(s * SUB + j, L)][0] - st
                    for cb in range(HID // L):
                        nb[j, pl.ds(cb * L, L)] = (
                            nb[j, pl.ds(cb * L, L)]
                            + pbuf[rel, pl.ds(cb * L, L)])

                return 0

            lax.fori_loop(0, SUB, add_body, 0)
        @pl.when(base + s * SUB < N_NODES)
        def _write(s=s):
            pltpu.async_copy(nbufs[s], out_hbm.at[pl.ds(base + s * SUB, SUB)],
                             sems_w[s])

        if s + 1 < NSUB:
            gp = pltpu.async_copy(ptab_hbm.at[pl.ds(starts[s + 1], PWIN)],
                                  pbuf, sem_p)
    with jax.named_scope("nk_drain"):
        for s in range(NSUB):
            @pl.when(base + s * SUB < N_NODES)
            def _drain(s=s):
                pltpu.make_async_copy(
                    nbufs[s], out_hbm.at[pl.ds(base + s * SUB, SUB)],
                    sems_w[s]).wait()


def _edge_body(ids_ref, tbl_ref, g_ref, b_ref, out_ref):
    tbl = tbl_ref[...]
    mean = jnp.mean(tbl, axis=1, keepdims=True)
    var = jnp.mean((tbl - mean) ** 2, axis=1, keepdims=True)
    norm = (tbl - mean) * lax.rsqrt(var + 1e-5) * g_ref[...] + b_ref[...]
    norm_hi = norm.astype(jnp.bfloat16)
    norm_lo = (norm - norm_hi.astype(jnp.float32)).astype(jnp.bfloat16)
    ids = ids_ref[0, 0, :]
    onehot = (ids[:, None] == lax.broadcasted_iota(jnp.int32, (BE, 64), 1)
              ).astype(jnp.bfloat16)
    out_ref[...] = (
        jnp.dot(onehot, norm_hi, preferred_element_type=jnp.float32)
        + jnp.dot(onehot, norm_lo, preferred_element_type=jnp.float32))


@jax.jit
def kernel(node_ids, top_mask, edge_ids, node_table, pos_table, edge_table,
           ln_gamma, ln_beta):
    node_ids = node_ids.astype(jnp.int32)
    ids2d = jnp.pad(node_ids, (0, NP - N_NODES)).reshape(NP // RPS, RPS)
    mask_pad = jnp.pad(top_mask.astype(jnp.int32), (0, NP - N_NODES))

    node_feat = pl.kernel(
        _node_body,
        out_type=jax.ShapeDtypeStruct((N_NODES, HID), jnp.float32),
        mesh=plsc.VectorSubcoreMesh(core_axis_name="c", subcore_axis_name="s",
                                    num_cores=NC, num_subcores=NS),
        scratch_types=[
            pltpu.VMEM((NSTR, RPS), jnp.int32),
            pltpu.VMEM((NP + L,), jnp.int32),
            pltpu.VMEM((C + L,), jnp.int32),
            [pltpu.VMEM((SUB, HID), jnp.float32)] * NSUB,
            pltpu.VMEM((PWIN, HID), jnp.float32),
            [pltpu.SemaphoreType.DMA] * NSUB,
            pltpu.SemaphoreType.DMA,
            [pltpu.SemaphoreType.DMA] * NSUB,
            pltpu.SemaphoreType.DMA,
        ],
        compiler_params=pltpu.CompilerParams(needs_layout_passes=False),
    )(ids2d, mask_pad, node_table, pos_table)

    edge_feat = pl.pallas_call(
        _edge_body,
        grid=(NB,),
        in_specs=[
            pl.BlockSpec((1, 1, BE), lambda i: (i, 0, 0)),
            pl.BlockSpec((64, HID), lambda i: (0, 0)),
            pl.BlockSpec((1, HID), lambda i: (0, 0)),
            pl.BlockSpec((1, HID), lambda i: (0, 0)),
        ],
        out_specs=pl.BlockSpec((BE, HID), lambda i: (i, 0)),
        out_shape=jax.ShapeDtypeStruct((N_EDGES, HID), jnp.float32),
    )(edge_ids.reshape(NB, 1, BE), edge_table,
      ln_gamma.reshape(1, HID), ln_beta.reshape(1, HID))

    return node_feat, edge_feat
